# baseline (speedup 1.0000x reference)
import jax
import jax.numpy as jnp
from jax import lax
from jax.experimental import pallas as pl
from jax.experimental.pallas import tpu as pltpu
from jax.experimental.pallas import tpu_sc as plsc

_BATCH, _SEQ, _EMBED = 4, 2048, 768
_NW = 32
_PPW = _SEQ // _NW
_CP = 16
_NCH = _PPW // _CP
_RPC = _BATCH * _CP
_NTB = 2
_NPB = 2
_LANES = 16
_VPR = _EMBED // _LANES
_GRP = 16


def _emb_body(x_hbm, tok_hbm, pos_hbm, out_hbm, idx_v,
              tbuf0, tbuf1, pbuf0, pbuf1, isem,
              gsem0, gsem1, psem0, psem1, wsem0, wsem1, hsem0, hsem1):
    tbufs = (tbuf0, tbuf1)
    pbufs = (pbuf0, pbuf1)
    gsems = (gsem0, gsem1)
    psems = (psem0, psem1)
    wsems = (wsem0, wsem1)
    hsems = (hsem0, hsem1)

    c = lax.axis_index("c")
    s = lax.axis_index("s")
    wid = s * 2 + c
    p0 = wid * _PPW

    def start_gather(ci, rb):
        return pltpu.async_copy(
            tok_hbm.at[idx_v.at[pl.ds(ci * _RPC, _RPC)]], tbufs[rb], gsems[rb]
        )

    def start_half_gather(ci, rb, half, sem):
        hw = _RPC // 2
        return pltpu.async_copy(
            tok_hbm.at[idx_v.at[pl.ds(ci * _RPC + half * hw, hw)]],
            tbufs[rb].at[pl.ds(half * hw, hw)],
            sem,
        )

    def start_pos(ci, rb):
        return pltpu.async_copy(
            pos_hbm.at[pl.ds(p0 + ci * _CP, _CP)], pbufs[rb], psems[rb]
        )

    def stage_idx(ci):
        return [
            pltpu.async_copy(
                x_hbm.at[pl.ds(b * _SEQ + p0 + ci * _CP, _CP)],
                idx_v.at[pl.ds(ci * _RPC + b * _CP, _CP)],
                isem,
            )
            for b in range(_BATCH)
        ]

    phandles = {ci: start_pos(ci, ci % _NPB) for ci in range(_NPB)}
    front = stage_idx(0) + stage_idx(1)
    rest = [h for ci in range(2, _NCH) for h in stage_idx(ci)]
    for h in front:
        h.wait()
    ghandles = {0: start_gather(0, 0), 1: start_gather(1, 1)}
    whandles = {}

    def make_pos_add(tb, pb, batches):
        def pos_add(i, carry):
            for g in range(_VPR // _GRP):
                pvecs = [
                    pbufs[pb][i, pl.ds((g * _GRP + k) * _LANES, _LANES)]
                    for k in range(_GRP)
                ]
                for b in batches:
                    row = b * _CP + i
                    for k in range(_GRP):
                        sl = pl.ds((g * _GRP + k) * _LANES, _LANES)
                        plsc.addupdate(tbufs[tb].at[row, sl], pvecs[k])
            return carry
        return pos_add

    def start_write(ci, tb, b):
        return pltpu.async_copy(
            tbufs[tb].at[pl.ds(b * _CP, _CP)],
            out_hbm.at[pl.ds(b * _SEQ + p0 + ci * _CP, _CP)],
            wsems[tb],
        )

    for ci in range(_NCH):
        tb = ci % _NTB
        pb = ci % _NPB
        if ci == _NCH - 1:
            ha, hb = ghandles.pop(ci)
            ha.wait()
            phandles.pop(ci).wait()
            whandles[ci] = []
            for b in (0, 1):
                lax.fori_loop(0, _CP, make_pos_add(tb, pb, (b,)), 0)
                whandles[ci].append(start_write(ci, tb, b))
            hb.wait()
            for b in (2, 3):
                lax.fori_loop(0, _CP, make_pos_add(tb, pb, (b,)), 0)
                whandles[ci].append(start_write(ci, tb, b))
        else:
            gh = ghandles.pop(ci)
            phandles.pop(ci).wait()
            if isinstance(gh, list):
                ha, hb = gh
                ha.wait()
                lax.fori_loop(0, _CP, make_pos_add(tb, pb, (0, 1)), 0)
                whandles[ci] = [start_write(ci, tb, 0), start_write(ci, tb, 1)]
                hb.wait()
                lax.fori_loop(0, _CP, make_pos_add(tb, pb, (2, 3)), 0)
                whandles[ci] += [start_write(ci, tb, 2), start_write(ci, tb, 3)]
            else:
                gh.wait()
                lax.fori_loop(0, _CP, make_pos_add(tb, pb, tuple(range(_BATCH))), 0)
                whandles[ci] = [start_write(ci, tb, b) for b in range(_BATCH)]

        if ci + _NPB < _NCH:
            phandles[ci + _NPB] = start_pos(ci + _NPB, pb)
        if ci + 2 < _NCH:
            if ci == 0:
                for h in rest:
                    h.wait()
            ws = whandles.pop(ci)
            ws[0].wait()
            ws[1].wait()
            ha = start_half_gather(ci + 2, tb, 0, gsems[tb])
            ws[2].wait()
            ws[3].wait()
            hb = start_half_gather(ci + 2, tb, 1, hsems[tb])
            ghandles[ci + 2] = [ha, hb]

    for ci in sorted(whandles):
        for h in whandles.pop(ci):
            h.wait()


@jax.jit
def kernel(x, tok_table, pos_table):
    xf = x.reshape(_BATCH * _SEQ)
    mesh = plsc.VectorSubcoreMesh(core_axis_name="c", subcore_axis_name="s")
    fn = pl.kernel(
        _emb_body,
        out_type=jax.ShapeDtypeStruct((_BATCH * _SEQ, _EMBED), jnp.float32),
        mesh=mesh,
        scratch_types=[
            pltpu.VMEM((_BATCH * _PPW,), jnp.int32),
            pltpu.VMEM((_RPC, _EMBED), jnp.float32),
            pltpu.VMEM((_RPC, _EMBED), jnp.float32),
            pltpu.VMEM((_CP, _EMBED), jnp.float32),
            pltpu.VMEM((_CP, _EMBED), jnp.float32),
            pltpu.SemaphoreType.DMA,
            pltpu.SemaphoreType.DMA,
            pltpu.SemaphoreType.DMA,
            pltpu.SemaphoreType.DMA,
            pltpu.SemaphoreType.DMA,
            pltpu.SemaphoreType.DMA,
            pltpu.SemaphoreType.DMA,
            pltpu.SemaphoreType.DMA,
            pltpu.SemaphoreType.DMA,
        ],
    )
    out = fn(xf, tok_table, pos_table)
    return out.reshape(_BATCH, _SEQ, _EMBED)


# speedup vs baseline: 1.0385x; 1.0385x over previous
import jax
import jax.numpy as jnp
from jax import lax
from jax.experimental import pallas as pl
from jax.experimental.pallas import tpu as pltpu
from jax.experimental.pallas import tpu_sc as plsc

_BATCH, _SEQ, _EMBED = 4, 2048, 768
_NW = 32
_PPW = _SEQ // _NW
_CP = 16
_NCH = _PPW // _CP
_RPC = _BATCH * _CP
_NTB = 2
_NPB = 2
_LANES = 16
_VPR = _EMBED // _LANES
_GRP = 16


def _emb_body(x_hbm, tok_hbm, pos_hbm, out_hbm, idx_v,
              tbuf0, tbuf1, pbuf0, pbuf1, isem,
              gsem0, gsem1, psem0, psem1, wsem0, wsem1, hsem0, hsem1):
    tbufs = (tbuf0, tbuf1)
    pbufs = (pbuf0, pbuf1)
    gsems = (gsem0, gsem1)
    psems = (psem0, psem1)
    wsems = (wsem0, wsem1)
    hsems = (hsem0, hsem1)

    c = lax.axis_index("c")
    s = lax.axis_index("s")
    wid = s * 2 + c
    p0 = wid * _PPW

    def start_gather(ci, rb):
        return pltpu.async_copy(
            tok_hbm.at[idx_v.at[pl.ds(ci * _RPC, _RPC)]], tbufs[rb], gsems[rb]
        )

    def start_half_gather(ci, rb, half, sem):
        hw = _RPC // 2
        return pltpu.async_copy(
            tok_hbm.at[idx_v.at[pl.ds(ci * _RPC + half * hw, hw)]],
            tbufs[rb].at[pl.ds(half * hw, hw)],
            sem,
        )

    def start_pos(ci, rb):
        return pltpu.async_copy(
            pos_hbm.at[pl.ds(p0 + ci * _CP, _CP)], pbufs[rb], psems[rb]
        )

    def stage_idx(ci):
        return [
            pltpu.async_copy(
                x_hbm.at[pl.ds(b * _SEQ + p0 + ci * _CP, _CP)],
                idx_v.at[pl.ds(ci * _RPC + b * _CP, _CP)],
                isem,
            )
            for b in range(_BATCH)
        ]

    phandles = {ci: start_pos(ci, ci % _NPB) for ci in range(_NPB)}
    front = stage_idx(0) + stage_idx(1)
    rest = [h for ci in range(2, _NCH) for h in stage_idx(ci)]
    for h in front:
        h.wait()
    ghandles = {0: start_gather(0, 0), 1: start_gather(1, 1)}
    whandles = {}

    def make_pos_add(tb, pb, batches):
        def pos_add(i, carry):
            for g in range(_VPR // _GRP):
                pvecs = [
                    pbufs[pb][i, pl.ds((g * _GRP + k) * _LANES, _LANES)]
                    for k in range(_GRP)
                ]
                for b in batches:
                    row = b * _CP + i
                    for k in range(_GRP):
                        sl = pl.ds((g * _GRP + k) * _LANES, _LANES)
                        plsc.addupdate(tbufs[tb].at[row, sl], pvecs[k])
            return carry
        return pos_add

    def start_write(ci, tb, b):
        return pltpu.async_copy(
            tbufs[tb].at[pl.ds(b * _CP, _CP)],
            out_hbm.at[pl.ds(b * _SEQ + p0 + ci * _CP, _CP)],
            wsems[tb],
        )

    for ci in range(_NCH):
        tb = ci % _NTB
        pb = ci % _NPB
        if ci == _NCH - 1:
            ha, hb = ghandles.pop(ci)
            ha.wait()
            phandles.pop(ci).wait()
            lax.fori_loop(0, _CP, make_pos_add(tb, pb, (0, 1)), 0)
            whandles[ci] = [start_write(ci, tb, 0), start_write(ci, tb, 1)]
            hb.wait()
            lax.fori_loop(0, _CP, make_pos_add(tb, pb, (2, 3)), 0)
            whandles[ci] += [start_write(ci, tb, 2), start_write(ci, tb, 3)]
        else:
            gh = ghandles.pop(ci)
            phandles.pop(ci).wait()
            if isinstance(gh, list):
                ha, hb = gh
                ha.wait()
                lax.fori_loop(0, _CP, make_pos_add(tb, pb, (0, 1)), 0)
                hb.wait()
                lax.fori_loop(0, _CP, make_pos_add(tb, pb, (2, 3)), 0)
            else:
                gh.wait()
                lax.fori_loop(0, _CP, make_pos_add(tb, pb, tuple(range(_BATCH))), 0)
            whandles[ci] = [start_write(ci, tb, b) for b in range(_BATCH)]

        if ci + _NPB < _NCH:
            phandles[ci + _NPB] = start_pos(ci + _NPB, pb)
        if ci + 2 < _NCH:
            if ci == 0:
                for h in rest:
                    h.wait()
            ws = whandles.pop(ci)
            ws[0].wait()
            ws[1].wait()
            ha = start_half_gather(ci + 2, tb, 0, gsems[tb])
            ws[2].wait()
            ws[3].wait()
            hb = start_half_gather(ci + 2, tb, 1, hsems[tb])
            ghandles[ci + 2] = [ha, hb]

    for ci in sorted(whandles):
        for h in whandles.pop(ci):
            h.wait()


@jax.jit
def kernel(x, tok_table, pos_table):
    xf = x.reshape(_BATCH * _SEQ)
    mesh = plsc.VectorSubcoreMesh(core_axis_name="c", subcore_axis_name="s")
    fn = pl.kernel(
        _emb_body,
        out_type=jax.ShapeDtypeStruct((_BATCH * _SEQ, _EMBED), jnp.float32),
        mesh=mesh,
        scratch_types=[
            pltpu.VMEM((_BATCH * _PPW,), jnp.int32),
            pltpu.VMEM((_RPC, _EMBED), jnp.float32),
            pltpu.VMEM((_RPC, _EMBED), jnp.float32),
            pltpu.VMEM((_CP, _EMBED), jnp.float32),
            pltpu.VMEM((_CP, _EMBED), jnp.float32),
            pltpu.SemaphoreType.DMA,
            pltpu.SemaphoreType.DMA,
            pltpu.SemaphoreType.DMA,
            pltpu.SemaphoreType.DMA,
            pltpu.SemaphoreType.DMA,
            pltpu.SemaphoreType.DMA,
            pltpu.SemaphoreType.DMA,
            pltpu.SemaphoreType.DMA,
            pltpu.SemaphoreType.DMA,
        ],
    )
    out = fn(xf, tok_table, pos_table)
    return out.reshape(_BATCH, _SEQ, _EMBED)
